# baseline (speedup 1.0000x reference)
import functools
import jax
import jax.numpy as jnp
from jax import lax
from jax.experimental import pallas as pl
from jax.experimental.pallas import tpu as pltpu
from jax.experimental.pallas import tpu_sc as plsc

_T, _C = 8192, 1024
_NC, _NS = 2, 16
_NW = _NC * _NS
_ROWS_PER_W = _T // _NW
_CHUNK = 32
_NCHUNKS = _ROWS_PER_W // _CHUNK


def _sc_copy_body(table_hbm, out_hbm, buf0, buf1, rs0, rs1, ws0, ws1):
    wid = lax.axis_index("s") * _NC + lax.axis_index("c")
    base = wid * _ROWS_PER_W
    bufs = (buf0, buf1)
    rsems = (rs0, rs1)
    wsems = (ws0, ws1)

    def rd(i, buf, sem):
        return pltpu.make_async_copy(
            table_hbm.at[pl.ds(base + i * _CHUNK, _CHUNK)], buf, sem)

    def wr(i, buf, sem):
        return pltpu.make_async_copy(
            buf, out_hbm.at[pl.ds(base + i * _CHUNK, _CHUNK)], sem)

    rd(0, bufs[0], rsems[0]).start()
    for i in range(_NCHUNKS):
        cur = i % 2
        nxt = 1 - cur
        if i + 1 < _NCHUNKS:
            if i >= 1:
                wr(i - 1, bufs[nxt], wsems[nxt]).wait()
            rd(i + 1, bufs[nxt], rsems[nxt]).start()
        rd(i, bufs[cur], rsems[cur]).wait()
        wr(i, bufs[cur], wsems[cur]).start()
    wr(_NCHUNKS - 2, bufs[(_NCHUNKS - 2) % 2], wsems[(_NCHUNKS - 2) % 2]).wait()
    wr(_NCHUNKS - 1, bufs[(_NCHUNKS - 1) % 2], wsems[(_NCHUNKS - 1) % 2]).wait()


@functools.cache
def _build_sc_copy():
    return pl.kernel(
        _sc_copy_body,
        mesh=plsc.VectorSubcoreMesh(core_axis_name="c", subcore_axis_name="s"),
        out_type=jax.ShapeDtypeStruct((_T, _C), jnp.float32),
        scratch_types=[
            pltpu.VMEM((_CHUNK, _C), jnp.float32),
            pltpu.VMEM((_CHUNK, _C), jnp.float32),
            pltpu.SemaphoreType.DMA,
            pltpu.SemaphoreType.DMA,
            pltpu.SemaphoreType.DMA,
            pltpu.SemaphoreType.DMA,
        ],
    )


def kernel(token_ids, table):
    return _build_sc_copy()(table)[None]
